# initial kernel scaffold (unmeasured)
import numpy as np
import jax
import jax.numpy as jnp
from jax import lax
from jax.experimental import pallas as pl
from jax.experimental.pallas import tpu as pltpu

N_DEV = 4
SQ = 1024
D = 1024
HQ = 8
DH = 128
SCALE = 0.08838834764831843

_inv = 1.0 / (10000.0 ** (np.arange(0, DH, 2) / DH))
_pos = np.arange(SQ)[:, None] * _inv[None, :]
_COS = np.repeat(np.cos(_pos), 2, axis=-1).astype(np.float32)
_SIN = np.repeat(np.sin(_pos), 2, axis=-1).astype(np.float32)


def kernel(x, Wq, Wk, Wv, Wo):

    def body(x_ref, wq_ref, wk_ref, wv_ref, wo_ref, cos_ref, sin_ref,
             out_ref, comm_ref, send_sems, recv_sems):
        my = lax.axis_index("i")
        left = lax.rem(my + (N_DEV - 1), N_DEV)
        right = lax.rem(my + 1, N_DEV)

        bar = pltpu.get_barrier_semaphore()
        for nbr in (left, right):
            pl.semaphore_signal(bar, inc=1, device_id=(nbr,),
                                device_id_type=pl.DeviceIdType.MESH)
        pl.semaphore_wait(bar, 2)

        xm = x_ref[0]
        cos = cos_ref[...]
        sin = sin_ref[...]
        col = lax.broadcasted_iota(jnp.int32, (SQ, DH), 1)
        even = (col % 2) == 0

        def rope(t):
            t_next = pltpu.roll(t, -1, 1)
            t_prev = pltpu.roll(t, 1, 1)
            rot = jnp.where(even, -t_next, t_prev)
            return t * cos + rot * sin

        acc = jnp.zeros((SQ, D), jnp.float32)
        for h in range(HQ):
            sl = pl.ds(h * DH, DH)
            q = rope(jnp.dot(xm, wq_ref[:, sl],
                             preferred_element_type=jnp.float32))
            k = rope(jnp.dot(xm, wk_ref[:, sl],
                             preferred_element_type=jnp.float32))
            v = jnp.dot(xm, wv_ref[:, sl],
                        preferred_element_type=jnp.float32)
            s = lax.dot_general(q, k, (((1,), (1,)), ((), ())),
                                preferred_element_type=jnp.float32) * SCALE
            m = jnp.max(s, axis=-1, keepdims=True)
            w = jnp.exp(s - m)
            w = w / jnp.sum(w, axis=-1, keepdims=True)
            ctx = jnp.dot(w, v, preferred_element_type=jnp.float32)
            acc = acc + jnp.dot(ctx, wo_ref[pl.ds(h * DH, DH), :],
                                preferred_element_type=jnp.float32)

        comm_ref[3] = acc
        for h in range(N_DEV - 1):
            rdma = pltpu.make_async_remote_copy(
                src_ref=comm_ref.at[3 - h],
                dst_ref=comm_ref.at[2 - h],
                send_sem=send_sems.at[h],
                recv_sem=recv_sems.at[h],
                device_id=(right,),
                device_id_type=pl.DeviceIdType.MESH,
            )
            rdma.start()
            rdma.wait()
            acc = acc + comm_ref[2 - h]

        out_ref[0] = acc

    cos = jnp.asarray(_COS)
    sin = jnp.asarray(_SIN)
    return pl.pallas_call(
        body,
        out_shape=jax.ShapeDtypeStruct((1, SQ, D), jnp.float32),
        in_specs=[pl.BlockSpec(memory_space=pltpu.VMEM)] * 7,
        out_specs=pl.BlockSpec(memory_space=pltpu.VMEM),
        scratch_shapes=[
            pltpu.VMEM((N_DEV, SQ, D), jnp.float32),
            pltpu.SemaphoreType.DMA((N_DEV - 1,)),
            pltpu.SemaphoreType.DMA((N_DEV - 1,)),
        ],
        compiler_params=pltpu.CompilerParams(collective_id=0),
    )(x, Wq, Wk, Wv, Wo, cos, sin)


# baseline (device time: 203569 ns/iter reference)
import numpy as np
import jax
import jax.numpy as jnp
from jax import lax
from jax.experimental import pallas as pl
from jax.experimental.pallas import tpu as pltpu

N_DEV = 4
SQ = 1024
D = 1024
HQ = 8
DH = 128
SCALE = 0.08838834764831843

_inv = 1.0 / (10000.0 ** (np.arange(0, DH, 2) / DH))
_pos = np.arange(SQ)[:, None] * _inv[None, :]
_COS = np.repeat(np.cos(_pos), 2, axis=-1).astype(np.float32)
_SIN = np.repeat(np.sin(_pos), 2, axis=-1).astype(np.float32)


def kernel(x, Wq, Wk, Wv, Wo):

    def body(x_ref, wq_ref, wk_ref, wv_ref, wo_ref, cos_ref, sin_ref,
             out_ref, comm_ref, send_sems, recv_sems):
        my = lax.axis_index("i")
        left = lax.rem(my + (N_DEV - 1), N_DEV)
        right = lax.rem(my + 1, N_DEV)

        bar = pltpu.get_barrier_semaphore()
        for nbr in (left, right):
            pl.semaphore_signal(bar, inc=1, device_id=(nbr,),
                                device_id_type=pl.DeviceIdType.MESH)
        pl.semaphore_wait(bar, 2)

        xm = x_ref[0]
        cos = cos_ref[...]
        sin = sin_ref[...]
        col = lax.broadcasted_iota(jnp.int32, (SQ, DH), 1)
        even = (col % 2) == 0

        def rope(t):
            t_next = pltpu.roll(t, DH - 1, 1)
            t_prev = pltpu.roll(t, 1, 1)
            rot = jnp.where(even, -t_next, t_prev)
            return t * cos + rot * sin

        acc = jnp.zeros((SQ, D), jnp.float32)
        for h in range(HQ):
            sl = pl.ds(h * DH, DH)
            q = rope(jnp.dot(xm, wq_ref[:, sl],
                             preferred_element_type=jnp.float32))
            k = rope(jnp.dot(xm, wk_ref[:, sl],
                             preferred_element_type=jnp.float32))
            v = jnp.dot(xm, wv_ref[:, sl],
                        preferred_element_type=jnp.float32)
            s = lax.dot_general(q, k, (((1,), (1,)), ((), ())),
                                preferred_element_type=jnp.float32) * SCALE
            m = jnp.max(s, axis=-1, keepdims=True)
            w = jnp.exp(s - m)
            w = w / jnp.sum(w, axis=-1, keepdims=True)
            ctx = jnp.dot(w, v, preferred_element_type=jnp.float32)
            acc = acc + jnp.dot(ctx, wo_ref[pl.ds(h * DH, DH), :],
                                preferred_element_type=jnp.float32)

        comm_ref[3] = acc
        for h in range(N_DEV - 1):
            rdma = pltpu.make_async_remote_copy(
                src_ref=comm_ref.at[3 - h],
                dst_ref=comm_ref.at[2 - h],
                send_sem=send_sems.at[h],
                recv_sem=recv_sems.at[h],
                device_id=(right,),
                device_id_type=pl.DeviceIdType.MESH,
            )
            rdma.start()
            rdma.wait()
            acc = acc + comm_ref[2 - h]

        out_ref[0] = acc

    cos = jnp.asarray(_COS)
    sin = jnp.asarray(_SIN)
    return pl.pallas_call(
        body,
        out_shape=jax.ShapeDtypeStruct((1, SQ, D), jnp.float32),
        in_specs=[pl.BlockSpec(memory_space=pltpu.VMEM)] * 7,
        out_specs=pl.BlockSpec(memory_space=pltpu.VMEM),
        scratch_shapes=[
            pltpu.VMEM((N_DEV, SQ, D), jnp.float32),
            pltpu.SemaphoreType.DMA((N_DEV - 1,)),
            pltpu.SemaphoreType.DMA((N_DEV - 1,)),
        ],
        compiler_params=pltpu.CompilerParams(collective_id=0),
    )(x, Wq, Wk, Wv, Wo, cos, sin)


# device time: 110325 ns/iter; 1.8452x vs baseline; 1.8452x over previous
import numpy as np
import jax
import jax.numpy as jnp
from jax import lax
from jax.experimental import pallas as pl
from jax.experimental.pallas import tpu as pltpu

N_DEV = 4
SQ = 1024
D = 1024
HQ = 8
DH = 128
CH = SQ // N_DEV
SCALE = 0.08838834764831843

_inv = 1.0 / (10000.0 ** (np.arange(0, DH, 2) / DH))
_pos = np.arange(SQ)[:, None] * _inv[None, :]
_COS = np.tile(np.repeat(np.cos(_pos), 2, axis=-1), (1, HQ)).astype(np.float32)
_SIN = np.tile(np.repeat(np.sin(_pos), 2, axis=-1), (1, HQ)).astype(np.float32)


def kernel(x, Wq, Wk, Wv, Wo):

    def body(x_ref, wq_ref, wk_ref, wv_ref, wo_ref, cos_ref, sin_ref,
             out_ref, k_ref, v_ref, part_ref, rs_ref, ag_ref,
             rs_send, rs_recv, ag_send, ag_recv):
        my = lax.axis_index("i")
        left = lax.rem(my + (N_DEV - 1), N_DEV)
        right = lax.rem(my + 1, N_DEV)

        bar = pltpu.get_barrier_semaphore()
        for nbr in (left, right):
            pl.semaphore_signal(bar, inc=1, device_id=(nbr,),
                                device_id_type=pl.DeviceIdType.MESH)
        pl.semaphore_wait(bar, 2)

        def rope(t, cosr, sinr):
            n = t.shape[1]
            even = (lax.broadcasted_iota(jnp.int32, t.shape, 1) % 2) == 0
            t_next = pltpu.roll(t, n - 1, 1)
            t_prev = pltpu.roll(t, 1, 1)
            return t * cosr + jnp.where(even, -t_next, t_prev) * sinr

        xm = x_ref[0]
        cos_f = cos_ref[...]
        sin_f = sin_ref[...]

        k_ref[...] = rope(jnp.dot(xm, wk_ref[...],
                                  preferred_element_type=jnp.float32),
                          cos_f, sin_f)
        v_ref[...] = jnp.dot(xm, wv_ref[...],
                             preferred_element_type=jnp.float32)

        def compute_chunk(rc):
            ro = rc * CH
            xq = x_ref[0, pl.ds(ro, CH), :]
            q = rope(jnp.dot(xq, wq_ref[...],
                             preferred_element_type=jnp.float32),
                     cos_ref[pl.ds(ro, CH), :], sin_ref[pl.ds(ro, CH), :])
            outc = jnp.zeros((CH, D), jnp.float32)
            for h in range(HQ):
                sl = pl.ds(h * DH, DH)
                s = lax.dot_general(q[:, h * DH:(h + 1) * DH], k_ref[:, sl],
                                    (((1,), (1,)), ((), ())),
                                    preferred_element_type=jnp.float32) * SCALE
                m = jnp.max(s, axis=-1, keepdims=True)
                w = jnp.exp(s - m)
                w = w / jnp.sum(w, axis=-1, keepdims=True)
                ctx = jnp.dot(w, v_ref[:, sl],
                              preferred_element_type=jnp.float32)
                outc = outc + jnp.dot(ctx, wo_ref[sl, :],
                                      preferred_element_type=jnp.float32)
            return outc

        def rs_copy(src, dst_slot, step):
            return pltpu.make_async_remote_copy(
                src_ref=src, dst_ref=rs_ref.at[dst_slot],
                send_sem=rs_send.at[step], recv_sem=rs_recv.at[step],
                device_id=(right,), device_id_type=pl.DeviceIdType.MESH)

        part_ref[0] = compute_chunk(my)
        rs0 = rs_copy(part_ref.at[0], 0, 0)
        rs0.start()

        part_ref[1] = compute_chunk(lax.rem(my + 3, N_DEV))
        rs0.wait_recv()
        rs_ref[0] = rs_ref[0] + part_ref[1]
        rs1 = rs_copy(rs_ref.at[0], 1, 1)
        rs1.start()

        part_ref[2] = compute_chunk(lax.rem(my + 2, N_DEV))
        rs1.wait_recv()
        rs_ref[1] = rs_ref[1] + part_ref[2]
        rs2 = rs_copy(rs_ref.at[1], 2, 2)
        rs2.start()

        part_ref[3] = compute_chunk(lax.rem(my + 1, N_DEV))
        rs2.wait_recv()
        owned = rs_ref[2] + part_ref[3]
        part_ref[3] = owned
        out_ref[0, pl.ds(lax.rem(my + 1, N_DEV) * CH, CH), :] = owned

        def ag_copy(src, dst_slot, hop):
            return pltpu.make_async_remote_copy(
                src_ref=src, dst_ref=ag_ref.at[dst_slot],
                send_sem=ag_send.at[hop], recv_sem=ag_recv.at[hop],
                device_id=(right,), device_id_type=pl.DeviceIdType.MESH)

        ag0 = ag_copy(part_ref.at[3], 0, 0)
        ag0.start()
        ag0.wait_recv()
        ag1 = ag_copy(ag_ref.at[0], 1, 1)
        ag1.start()
        out_ref[0, pl.ds(my * CH, CH), :] = ag_ref[0]
        ag1.wait_recv()
        ag2 = ag_copy(ag_ref.at[1], 2, 2)
        ag2.start()
        out_ref[0, pl.ds(lax.rem(my + 3, N_DEV) * CH, CH), :] = ag_ref[1]
        ag2.wait_recv()
        out_ref[0, pl.ds(lax.rem(my + 2, N_DEV) * CH, CH), :] = ag_ref[2]

        for d in (rs0, rs1, rs2, ag0, ag1, ag2):
            d.wait_send()

    cos = jnp.asarray(_COS)
    sin = jnp.asarray(_SIN)
    return pl.pallas_call(
        body,
        out_shape=jax.ShapeDtypeStruct((1, SQ, D), jnp.float32),
        in_specs=[pl.BlockSpec(memory_space=pltpu.VMEM)] * 7,
        out_specs=pl.BlockSpec(memory_space=pltpu.VMEM),
        scratch_shapes=[
            pltpu.VMEM((SQ, D), jnp.float32),
            pltpu.VMEM((SQ, D), jnp.float32),
            pltpu.VMEM((N_DEV, CH, D), jnp.float32),
            pltpu.VMEM((N_DEV - 1, CH, D), jnp.float32),
            pltpu.VMEM((N_DEV - 1, CH, D), jnp.float32),
            pltpu.SemaphoreType.DMA((N_DEV - 1,)),
            pltpu.SemaphoreType.DMA((N_DEV - 1,)),
            pltpu.SemaphoreType.DMA((N_DEV - 1,)),
            pltpu.SemaphoreType.DMA((N_DEV - 1,)),
        ],
        compiler_params=pltpu.CompilerParams(
            collective_id=0, vmem_limit_bytes=100 * 1024 * 1024),
    )(x, Wq, Wk, Wv, Wo, cos, sin)
